# baseline (speedup 1.0000x reference)
import jax
import jax.numpy as jnp
from jax.experimental import pallas as pl
from jax.experimental.pallas import tpu as pltpu

_N_CHUNKS = 8


def _dma_kernel(src_ref, dst_ref, sems):
    rows = src_ref.shape[0] // _N_CHUNKS
    copies = []
    for i in range(_N_CHUNKS):
        sl = pl.ds(i * rows, rows)
        c = pltpu.make_async_copy(src_ref.at[sl], dst_ref.at[sl], sems.at[i])
        c.start()
        copies.append(c)
    for c in copies:
        c.wait()


def kernel(node_features, edges, layer_lengths, W1, b1, W2, b2):
    n = edges.shape[0] * edges.shape[1]
    flat = edges.reshape(n // 128, 128)
    out = pl.pallas_call(
        _dma_kernel,
        in_specs=[pl.BlockSpec(memory_space=pl.ANY)],
        out_specs=pl.BlockSpec(memory_space=pl.ANY),
        out_shape=jax.ShapeDtypeStruct(flat.shape, flat.dtype),
        scratch_shapes=[pltpu.SemaphoreType.DMA((_N_CHUNKS,))],
    )(flat)
    return out.reshape(edges.shape)


# speedup vs baseline: 3.6537x; 3.6537x over previous
import jax
import jax.numpy as jnp
from jax.experimental import pallas as pl
from jax.experimental.pallas import tpu as pltpu

_GRID = 25


def _copy_kernel(src_ref, dst_ref):
    dst_ref[...] = src_ref[...]


def kernel(node_features, edges, layer_lengths, W1, b1, W2, b2):
    n = edges.shape[0] * edges.shape[1]
    rows = n // 128
    flat = edges.reshape(rows, 128)
    blk = rows // _GRID
    out = pl.pallas_call(
        _copy_kernel,
        grid=(_GRID,),
        in_specs=[pl.BlockSpec((blk, 128), lambda i: (i, 0))],
        out_specs=pl.BlockSpec((blk, 128), lambda i: (i, 0)),
        out_shape=jax.ShapeDtypeStruct(flat.shape, flat.dtype),
    )(flat)
    return out.reshape(edges.shape)


# speedup vs baseline: 6.6516x; 1.8205x over previous
import jax
import jax.numpy as jnp
from jax.experimental import pallas as pl
from jax.experimental.pallas import tpu as pltpu


def _copy_kernel(src_ref, dst_ref):
    dst_ref[...] = src_ref[...]


def kernel(node_features, edges, layer_lengths, W1, b1, W2, b2):
    n = edges.shape[0] * edges.shape[1]
    flat = edges.reshape(n // 128, 128)
    out = pl.pallas_call(
        _copy_kernel,
        out_shape=jax.ShapeDtypeStruct(flat.shape, flat.dtype),
    )(flat)
    return out.reshape(edges.shape)
